# Initial kernel scaffold; baseline (speedup 1.0000x reference)
import jax
import jax.numpy as jnp
from jax.experimental import pallas as pl


def kernel(x):
    raise NotImplementedError("write your pallas kernel here")


# speedup vs baseline: 22.1637x; 22.1637x over previous
import functools

import jax
import jax.numpy as jnp
from jax.experimental import pallas as pl

TOPK = 64
C = 768


def _topk_mask_kernel(x_ref, o_ref):
    x = x_ref[0]
    i = jax.lax.bitcast_convert_type(x, jnp.int32)
    s = i ^ ((i >> 31) & jnp.int32(0x7FFFFFFF))
    hw = x.shape[1]
    p = jnp.full((1, hw), jnp.iinfo(jnp.int32).min, dtype=jnp.int32)
    for bit in range(31, -1, -1):
        step = jnp.int32(-(2 ** 31)) if bit == 31 else jnp.int32(1 << bit)
        q = p + step
        cnt = jnp.sum((s >= q).astype(jnp.float32), axis=0, keepdims=True)
        p = jnp.where(cnt >= TOPK, q, p)
    o_ref[0] = jnp.where(s >= p, x, jnp.float32(0.0))


def _run(x3, hw):
    b = x3.shape[0]
    return pl.pallas_call(
        _topk_mask_kernel,
        grid=(b,),
        in_specs=[pl.BlockSpec((1, C, hw), lambda ib: (ib, 0, 0))],
        out_specs=pl.BlockSpec((1, C, hw), lambda ib: (ib, 0, 0)),
        out_shape=jax.ShapeDtypeStruct(x3.shape, x3.dtype),
    )(x3)


def kernel(x):
    B, c, H, W = x.shape
    x3 = x.reshape(B, c, H * W)
    out = _run(x3, H * W)
    return out.reshape(B, c, H, W)


# speedup vs baseline: 28.7361x; 1.2965x over previous
import jax
import jax.numpy as jnp
from jax.experimental import pallas as pl

TOPK = 64
C = 768
I16_MIN = -(2 ** 15)
I16_MAX = 2 ** 15 - 1


def _count_ge(vals, q):
    m = (vals >= q).astype(jnp.int16)
    r = m.shape[0]
    while r > 3:
        half = r // 2
        m = m[:half] + m[half:]
        r = half
    return m[0:1] + m[1:2] + m[2:3]


def _radix16(vals, k):
    hw = vals.shape[1]
    p = jnp.full((1, hw), I16_MIN, dtype=jnp.int16)
    for bit in range(15, -1, -1):
        step = jnp.int16(I16_MIN) if bit == 15 else jnp.int16(1 << bit)
        q = p + step
        cnt = _count_ge(vals, q)
        p = jnp.where(cnt >= k, q, p)
    return p


def _topk_mask_kernel(x_ref, o_ref):
    x = x_ref[0]
    i = jax.lax.bitcast_convert_type(x, jnp.int32)
    s = i ^ ((i >> 31) & jnp.int32(0x7FFFFFFF))
    hw = x.shape[1]

    s_hi = (s >> 16).astype(jnp.int16)
    k1 = jnp.full((1, hw), TOPK, dtype=jnp.int16)
    h = _radix16(s_hi, k1)

    c_gt = _count_ge(s_hi, h + jnp.int16(1))
    c_gt = jnp.where(h == jnp.int16(I16_MAX), jnp.int16(0), c_gt)
    lo = ((s & jnp.int32(0xFFFF)) ^ jnp.int32(0x8000)).astype(jnp.int16)
    lo = jnp.where(s_hi == h, lo, jnp.int16(I16_MIN))
    p2 = _radix16(lo, k1 - c_gt)

    p32 = (h.astype(jnp.int32) << 16) | (
        (p2.astype(jnp.int32) ^ jnp.int32(0x8000)) & jnp.int32(0xFFFF))
    o_ref[0] = jnp.where(s >= p32, x, jnp.float32(0.0))


def _run(x3, hw):
    b = x3.shape[0]
    return pl.pallas_call(
        _topk_mask_kernel,
        grid=(b,),
        in_specs=[pl.BlockSpec((1, C, hw), lambda ib: (ib, 0, 0))],
        out_specs=pl.BlockSpec((1, C, hw), lambda ib: (ib, 0, 0)),
        out_shape=jax.ShapeDtypeStruct(x3.shape, x3.dtype),
    )(x3)


def kernel(x):
    B, c, H, W = x.shape
    x3 = x.reshape(B, c, H * W)
    out = _run(x3, H * W)
    return out.reshape(B, c, H, W)


# speedup vs baseline: 30.2164x; 1.0515x over previous
import jax
import jax.numpy as jnp
from jax.experimental import pallas as pl

TOPK = 64
C = 768
I16_MIN = -(2 ** 15)
I16_MAX = 2 ** 15 - 1


CHUNK = 128


def _count_ge(vals, q):
    r = vals.shape[0]
    if r <= CHUNK:
        m = (vals >= q).astype(jnp.int16)
    else:
        m = (vals[0:CHUNK] >= q).astype(jnp.int16)
        for c in range(CHUNK, r, CHUNK):
            m = m + (vals[c:c + CHUNK] >= q).astype(jnp.int16)
        r = CHUNK
    while r > 1 and r % 2 == 0:
        half = r // 2
        m = m[:half] + m[half:]
        r = half
    if r == 3:
        return m[0:1] + m[1:2] + m[2:3]
    return m[0:1]


def _radix16(vals, k):
    hw = vals.shape[1]
    p = jnp.full((1, hw), I16_MIN, dtype=jnp.int16)
    for bit in range(15, -1, -1):
        step = jnp.int16(I16_MIN) if bit == 15 else jnp.int16(1 << bit)
        q = p + step
        cnt = _count_ge(vals, q)
        p = jnp.where(cnt >= k, q, p)
    return p


def _topk_mask_kernel(x_ref, o_ref):
    x = x_ref[0]
    i = jax.lax.bitcast_convert_type(x, jnp.int32)
    s = i ^ ((i >> 31) & jnp.int32(0x7FFFFFFF))
    hw = x.shape[1]

    s_hi = (s >> 16).astype(jnp.int16)
    k1 = jnp.full((1, hw), TOPK, dtype=jnp.int16)
    h = _radix16(s_hi, k1)

    c_gt = _count_ge(s_hi, h + jnp.int16(1))
    c_gt = jnp.where(h == jnp.int16(I16_MAX), jnp.int16(0), c_gt)
    lo = ((s & jnp.int32(0xFFFF)) ^ jnp.int32(0x8000)).astype(jnp.int16)
    lo = jnp.where(s_hi == h, lo, jnp.int16(I16_MIN))
    p2 = _radix16(lo, k1 - c_gt)

    p32 = (h.astype(jnp.int32) << 16) | (
        (p2.astype(jnp.int32) ^ jnp.int32(0x8000)) & jnp.int32(0xFFFF))
    o_ref[0] = jnp.where(s >= p32, x, jnp.float32(0.0))


def _run(x3, hw):
    b = x3.shape[0]
    return pl.pallas_call(
        _topk_mask_kernel,
        grid=(b,),
        in_specs=[pl.BlockSpec((1, C, hw), lambda ib: (ib, 0, 0))],
        out_specs=pl.BlockSpec((1, C, hw), lambda ib: (ib, 0, 0)),
        out_shape=jax.ShapeDtypeStruct(x3.shape, x3.dtype),
    )(x3)


def kernel(x):
    B, c, H, W = x.shape
    x3 = x.reshape(B, c, H * W)
    out = _run(x3, H * W)
    return out.reshape(B, c, H, W)
